# baseline (speedup 1.0000x reference)
import functools

import jax
import jax.numpy as jnp
from jax import lax
from jax.experimental import pallas as pl
from jax.experimental.pallas import tpu as pltpu
from jax.experimental.pallas import tpu_sc as plsc

N, C, S = 2048, 8, 2048
L = 16
NCORES = 2
NSUB = 16
CLS_PER_CORE = C // NCORES
CHUNKS = S // L
PBLK = 8

SC_P = 256
SC_NQ = 256
SPAN_P = SC_P // NSUB
SPAN_Q = SC_NQ // NSUB
RB = 128

EPS = 1.1920928955078125e-07
SCALE = float(100.0 / (N * S + EPS))

_i32 = jnp.int32


def _row_argmax(buf, r, iota, neg_inf, zeros_i):
    def chunk_body(_, carry):
        maxv, maxk, k = carry
        v = buf[_i32(r), pl.ds(k * _i32(L), L)]
        m = v > maxv
        return (jnp.where(m, v, maxv), jnp.where(m, k, maxk), k + _i32(1))

    maxv, maxk, _ = lax.fori_loop(0, CHUNKS, chunk_body,
                                  (neg_inf, zeros_i, _i32(0)), unroll=8)
    mval = jnp.max(maxv)
    cand = jnp.where(maxv == mval, maxk * _i32(L) + iota, _i32(S))
    return jnp.min(cand)


def _argmax_body(pred_hbm, lab_hbm, pbuf0, pbuf1, labbuf, psem0, psem1):
    core = lax.axis_index("c")
    sid = lax.axis_index("s")
    iota = lax.iota(jnp.int32, L)
    neg_inf = jnp.full((L,), -jnp.inf, dtype=jnp.float32)
    zeros_i = jnp.zeros((L,), dtype=jnp.int32)
    s0 = sid * _i32(SPAN_P)

    for cl in range(CLS_PER_CORE):
        c = core * _i32(CLS_PER_CORE) + _i32(cl)

        def pstart(blk, buf, sem, c=c):
            base = jnp.minimum(s0 + blk * _i32(PBLK), _i32(N - PBLK))
            pltpu.async_copy(pred_hbm.at[pl.ds(base, PBLK), c], buf, sem)

        def pwait(sem):
            pltpu.make_async_copy(
                pred_hbm.at[pl.ds(_i32(0), PBLK), _i32(0)], pbuf0, sem).wait()

        pstart(_i32(0), pbuf0, psem0)

        def pgrp(g, _, c=c):
            pstart(_i32(2) * g + _i32(1), pbuf1, psem1, c=c)
            pwait(psem0)
            lab_vec = zeros_i
            for r in range(PBLK):
                lab = _row_argmax(pbuf0, r, iota, neg_inf, zeros_i)
                lab_vec = jnp.where(iota == _i32(r), lab, lab_vec)
            pstart(_i32(2) * g + _i32(2), pbuf0, psem0, c=c)
            pwait(psem1)
            for r in range(PBLK):
                lab = _row_argmax(pbuf1, r, iota, neg_inf, zeros_i)
                lab_vec = jnp.where(iota == _i32(PBLK + r), lab, lab_vec)
            labbuf[pl.ds(g * _i32(L), L)] = lab_vec
            return _i32(0)

        lax.fori_loop(_i32(0), _i32(SPAN_P // (2 * PBLK)), pgrp, _i32(0))
        pwait(psem0)
        pltpu.sync_copy(labbuf, lab_hbm.at[c, pl.ds(s0, SPAN_P)])


def _count_block(buf, labtile, acc):
    def chunk_body(_, carry):
        a0, a1, k = carry
        off = k * _i32(L)
        lab = labtile[pl.ds(off, L)]
        for r in range(PBLK):
            eq = (buf[_i32(r), pl.ds(off, L)] == lab).astype(jnp.int32)
            if r % 2 == 0:
                a0 = a0 + eq
            else:
                a1 = a1 + eq
        return (a0, a1, k + _i32(1))

    a0, a1, _ = lax.fori_loop(0, CHUNKS, chunk_body, (*acc, _i32(0)),
                              unroll=2)
    return (a0, a1)


def _count_body(target_hbm, lab_hbm, cnt_hbm,
                tbuf0, tbuf1, labtile, cntbuf, tsem0, tsem1):
    core = lax.axis_index("c")
    sid = lax.axis_index("s")
    iota = lax.iota(jnp.int32, L)
    zeros_i = jnp.zeros((L,), dtype=jnp.int32)
    n0 = sid * _i32(SPAN_Q)

    cnt_vec = zeros_i
    for cl in range(CLS_PER_CORE):
        c = core * _i32(CLS_PER_CORE) + _i32(cl)
        pltpu.sync_copy(lab_hbm.at[c], labtile)

        def tstart(blk, buf, sem, c=c):
            base = jnp.minimum(n0 + blk * _i32(PBLK), _i32(N - PBLK))
            pltpu.async_copy(target_hbm.at[pl.ds(base, PBLK), c], buf, sem)

        def twait(sem):
            pltpu.make_async_copy(
                target_hbm.at[pl.ds(_i32(0), PBLK), _i32(0)],
                tbuf0, sem).wait()

        tstart(_i32(0), tbuf0, tsem0)

        def tgrp(g, acc, c=c):
            tstart(_i32(2) * g + _i32(1), tbuf1, tsem1, c=c)
            twait(tsem0)
            acc = _count_block(tbuf0, labtile, acc)
            tstart(_i32(2) * g + _i32(2), tbuf0, tsem0, c=c)
            twait(tsem1)
            acc = _count_block(tbuf1, labtile, acc)
            return acc

        a0, a1 = lax.fori_loop(_i32(0), _i32(SPAN_Q // (2 * PBLK)), tgrp,
                               (zeros_i, zeros_i))
        twait(tsem0)
        cnt_vec = jnp.where(iota == _i32(cl),
                            jnp.sum(a0 + a1, dtype=jnp.int32), cnt_vec)

    cntbuf[...] = cnt_vec
    pltpu.sync_copy(cntbuf, cnt_hbm.at[core, sid])


def _mesh():
    return plsc.VectorSubcoreMesh(core_axis_name="c", subcore_axis_name="s")


def _tc_argmax(pred_ref, lab_ref):
    x = pred_ref[...]
    m = jnp.max(x, axis=-1, keepdims=True)
    idx = lax.broadcasted_iota(jnp.int32, x.shape, 2)
    first = jnp.min(jnp.where(x == m, idx, _i32(S)), axis=-1)
    lab_ref[...] = first


def _tc_count(lab_ref, targ_ref, out_ref):
    b = pl.program_id(0)

    @pl.when(b == 0)
    def _():
        out_ref[...] = jnp.zeros((C, S), dtype=jnp.int32)

    t = targ_ref[...]
    lab = lab_ref[...]
    eq = (t == lab[None]).astype(jnp.int32)
    out_ref[...] += jnp.sum(eq, axis=0, dtype=jnp.int32)


def _tc_combine(sc_cnt_ref, tc_cnt_ref, out_ref):
    sc = sc_cnt_ref[...].astype(jnp.float32)
    tot_sc = jnp.sum(sc, axis=1)[:, :CLS_PER_CORE]
    tc = tc_cnt_ref[...].astype(jnp.float32)
    tot_tc = jnp.sum(tc.reshape(NCORES, CLS_PER_CORE, S), axis=-1)
    out_ref[...] = (tot_sc + tot_tc + EPS) * SCALE


@jax.jit
def _accuracy(pred, target):
    params = pltpu.CompilerParams(needs_layout_passes=False)

    sc_lab = functools.partial(
        pl.kernel,
        out_type=jax.ShapeDtypeStruct((C, SC_P), jnp.int32),
        mesh=_mesh(),
        compiler_params=params,
        scratch_types=[
            pltpu.VMEM((PBLK, S), jnp.float32),
            pltpu.VMEM((PBLK, S), jnp.float32),
            pltpu.VMEM((SPAN_P,), jnp.int32),
            pltpu.SemaphoreType.DMA,
            pltpu.SemaphoreType.DMA,
        ],
    )(_argmax_body)(pred)

    tc_lab = pl.pallas_call(
        _tc_argmax,
        grid=((N - SC_P) // RB,),
        in_specs=[pl.BlockSpec((RB, C, S),
                               lambda b: (b + SC_P // RB, _i32(0), _i32(0)))],
        out_specs=pl.BlockSpec((RB, C), lambda b: (b, _i32(0))),
        out_shape=jax.ShapeDtypeStruct((N - SC_P, C), jnp.int32),
    )(pred)

    lab_cs = jnp.concatenate([sc_lab, tc_lab.T], axis=1)

    sc_cnt = functools.partial(
        pl.kernel,
        out_type=jax.ShapeDtypeStruct((NCORES, NSUB, L), jnp.int32),
        mesh=_mesh(),
        compiler_params=params,
        scratch_types=[
            pltpu.VMEM((PBLK, S), jnp.int32),
            pltpu.VMEM((PBLK, S), jnp.int32),
            pltpu.VMEM((S,), jnp.int32),
            pltpu.VMEM((L,), jnp.int32),
            pltpu.SemaphoreType.DMA,
            pltpu.SemaphoreType.DMA,
        ],
    )(_count_body)(target, lab_cs)

    tc_cnt = pl.pallas_call(
        _tc_count,
        grid=((N - SC_NQ) // RB,),
        in_specs=[
            pl.BlockSpec((C, S), lambda b: (_i32(0), _i32(0))),
            pl.BlockSpec((RB, C, S),
                         lambda b: (b + SC_NQ // RB, _i32(0), _i32(0))),
        ],
        out_specs=pl.BlockSpec((C, S), lambda b: (_i32(0), _i32(0))),
        out_shape=jax.ShapeDtypeStruct((C, S), jnp.int32),
    )(lab_cs, target)

    return pl.pallas_call(
        _tc_combine,
        out_shape=jax.ShapeDtypeStruct((NCORES, CLS_PER_CORE), jnp.float32),
    )(sc_cnt, tc_cnt)


def kernel(pred, target):
    target = target.astype(jnp.int32)
    return _accuracy(pred, target).reshape(C, 1)


# speedup vs baseline: 1.0125x; 1.0125x over previous
import functools

import jax
import jax.numpy as jnp
from jax import lax
from jax.experimental import pallas as pl
from jax.experimental.pallas import tpu as pltpu
from jax.experimental.pallas import tpu_sc as plsc

N, C, S = 2048, 8, 2048
L = 16
NCORES = 2
NSUB = 16
CLS_PER_CORE = C // NCORES
CHUNKS = S // L
PBLK = 8

P = 1024
SPAN_P = P // NSUB
RB = 128
RBC = 256

EPS = 1.1920928955078125e-07
SCALE = float(100.0 / (N * S + EPS))

_i32 = jnp.int32


def _row_argmax(buf, r, iota, neg_inf, zeros_i):
    def chunk_body(_, carry):
        maxv, maxk, k = carry
        v = buf[_i32(r), pl.ds(k * _i32(L), L)]
        m = v > maxv
        return (jnp.where(m, v, maxv), jnp.where(m, k, maxk), k + _i32(1))

    maxv, maxk, _ = lax.fori_loop(0, CHUNKS, chunk_body,
                                  (neg_inf, zeros_i, _i32(0)), unroll=8)
    mval = jnp.max(maxv)
    cand = jnp.where(maxv == mval, maxk * _i32(L) + iota, _i32(S))
    return jnp.min(cand)


def _argmax_body(pred_hbm, lab_hbm, pbuf0, pbuf1, labbuf, psem0, psem1):
    core = lax.axis_index("c")
    sid = lax.axis_index("s")
    iota = lax.iota(jnp.int32, L)
    neg_inf = jnp.full((L,), -jnp.inf, dtype=jnp.float32)
    zeros_i = jnp.zeros((L,), dtype=jnp.int32)
    s0 = sid * _i32(SPAN_P)

    for cl in range(CLS_PER_CORE):
        c = core * _i32(CLS_PER_CORE) + _i32(cl)

        def pstart(blk, buf, sem, c=c):
            base = jnp.minimum(s0 + blk * _i32(PBLK), _i32(N - PBLK))
            pltpu.async_copy(pred_hbm.at[pl.ds(base, PBLK), c], buf, sem)

        def pwait(sem):
            pltpu.make_async_copy(
                pred_hbm.at[pl.ds(_i32(0), PBLK), _i32(0)], pbuf0, sem).wait()

        pstart(_i32(0), pbuf0, psem0)

        def pgrp(g, _, c=c):
            pstart(_i32(2) * g + _i32(1), pbuf1, psem1, c=c)
            pwait(psem0)
            lab_vec = zeros_i
            for r in range(PBLK):
                lab = _row_argmax(pbuf0, r, iota, neg_inf, zeros_i)
                lab_vec = jnp.where(iota == _i32(r), lab, lab_vec)
            pstart(_i32(2) * g + _i32(2), pbuf0, psem0, c=c)
            pwait(psem1)
            for r in range(PBLK):
                lab = _row_argmax(pbuf1, r, iota, neg_inf, zeros_i)
                lab_vec = jnp.where(iota == _i32(PBLK + r), lab, lab_vec)
            labbuf[pl.ds(g * _i32(L), L)] = lab_vec
            return _i32(0)

        lax.fori_loop(_i32(0), _i32(SPAN_P // (2 * PBLK)), pgrp, _i32(0))
        pwait(psem0)
        pltpu.sync_copy(labbuf, lab_hbm.at[c, pl.ds(s0, SPAN_P)])


def _mesh():
    return plsc.VectorSubcoreMesh(core_axis_name="c", subcore_axis_name="s")


def _tc_argmax(pred_ref, lab_ref):
    x = pred_ref[...]
    m = jnp.max(x, axis=-1, keepdims=True)
    idx = lax.broadcasted_iota(jnp.int32, x.shape, 2)
    first = jnp.min(jnp.where(x == m, idx, _i32(S)), axis=-1)
    lab_ref[...] = first


def _tc_count(lab_ref, targ_ref, out_ref):
    b = pl.program_id(0)

    @pl.when(b == 0)
    def _():
        out_ref[...] = jnp.zeros((C, P), dtype=jnp.int32)

    t = targ_ref[...]
    lab = lab_ref[...]
    eq = (t == lab[None]).astype(jnp.int32)
    out_ref[...] += jnp.sum(eq, axis=0, dtype=jnp.int32)


def _tc_combine(lo_ref, hi_ref, out_ref):
    lo = lo_ref[...].astype(jnp.float32)
    hi = hi_ref[...].astype(jnp.float32)
    tot = (jnp.sum(lo.reshape(NCORES, CLS_PER_CORE, P), axis=-1)
           + jnp.sum(hi.reshape(NCORES, CLS_PER_CORE, P), axis=-1))
    out_ref[...] = (tot + EPS) * SCALE


def _count_call(lab, target, lane_block):
    return pl.pallas_call(
        _tc_count,
        grid=(N // RBC,),
        in_specs=[
            pl.BlockSpec((C, P), lambda b: (_i32(0), _i32(0))),
            pl.BlockSpec((RBC, C, P),
                         lambda b: (b, _i32(0), _i32(lane_block))),
        ],
        out_specs=pl.BlockSpec((C, P), lambda b: (_i32(0), _i32(0))),
        out_shape=jax.ShapeDtypeStruct((C, P), jnp.int32),
    )(lab, target)


@jax.jit
def _accuracy(pred, target):
    sc_lab = functools.partial(
        pl.kernel,
        out_type=jax.ShapeDtypeStruct((C, P), jnp.int32),
        mesh=_mesh(),
        compiler_params=pltpu.CompilerParams(needs_layout_passes=False),
        scratch_types=[
            pltpu.VMEM((PBLK, S), jnp.float32),
            pltpu.VMEM((PBLK, S), jnp.float32),
            pltpu.VMEM((SPAN_P,), jnp.int32),
            pltpu.SemaphoreType.DMA,
            pltpu.SemaphoreType.DMA,
        ],
    )(_argmax_body)(pred)

    tc_lab = pl.pallas_call(
        _tc_argmax,
        grid=((N - P) // RB,),
        in_specs=[pl.BlockSpec((RB, C, S),
                               lambda b: (b + P // RB, _i32(0), _i32(0)))],
        out_specs=pl.BlockSpec((RB, C), lambda b: (b, _i32(0))),
        out_shape=jax.ShapeDtypeStruct((N - P, C), jnp.int32),
    )(pred)

    cnt_hi = _count_call(tc_lab.T, target, 1)
    cnt_lo = _count_call(sc_lab, target, 0)

    return pl.pallas_call(
        _tc_combine,
        out_shape=jax.ShapeDtypeStruct((NCORES, CLS_PER_CORE), jnp.float32),
    )(cnt_lo, cnt_hi)


def kernel(pred, target):
    target = target.astype(jnp.int32)
    return _accuracy(pred, target).reshape(C, 1)
